# baseline (speedup 1.0000x reference)
import functools

import jax
import jax.numpy as jnp
from jax import lax
from jax.experimental import pallas as pl
from jax.experimental.pallas import tpu as pltpu
from jax.experimental.pallas import tpu_sc as plsc

B, H, W, N, M = 16, 128, 128, 512, 512
HW = H * W
NBC = 8

F = jnp.float32
C041 = 0.41
OFFY = (0.0, 1.0, 0.0, 1.0)
OFFX = (0.0, 0.0, 1.0, 1.0)


def _iou16(w0, y0, x0, w1, y1, x1):
    a0 = (w0 * w0) * C041
    a1 = (w1 * w1) * C041
    ay0 = y0 - w0 * 0.5
    ay1 = y0 + w0 * 0.5
    ax0 = x0 - (C041 * w0) * 0.5
    ax1 = x0 + (C041 * w0) * 0.5
    by0 = y1 - w1 * 0.5
    by1 = y1 + w1 * 0.5
    bx0 = x1 - (C041 * w1) * 0.5
    bx1 = x1 + (C041 * w1) * 0.5
    iy = jnp.maximum(jnp.minimum(ay1, by1) - jnp.maximum(ay0, by0), 0.0)
    ix = jnp.maximum(jnp.minimum(ax1, bx1) - jnp.maximum(ax0, bx0), 0.0)
    inter = iy * ix
    union = a0 + a1 - inter
    return inter / (union + 1e-6)


def _sc_body(h_hbm, off_hbm, aidx_hbm, ridx_hbm, am_hbm, rm_hbm, pre_hbm,
             out_hbm, h_v, off_v, aidx_v, ridx_v, am_v, rm_v, pre_v, stage_v,
             buf_v, shared_v, sem_a, sem_r):
    c = lax.axis_index("c")
    s = lax.axis_index("s")
    active = s < NBC
    b = jnp.where(active, c * NBC + s, 0)

    la = lax.iota(jnp.int32, 16)
    zero = jnp.zeros((16,), F)
    czero = jnp.zeros((16,), jnp.int32)
    cone = czero + 1

    acc_a = acc_an = acc_r = acc_rn = zero

    @pl.when(active)
    def _work():
        group_a = [
            pltpu.async_copy(h_hbm.at[b, 0], h_v, sem_a),
            pltpu.async_copy(off_hbm.at[b], off_v, sem_a),
            pltpu.async_copy(aidx_hbm.at[b], aidx_v, sem_a),
            pltpu.async_copy(am_hbm.at[b], am_v, sem_a),
        ]
        group_r = [
            pltpu.async_copy(ridx_hbm.at[b], ridx_v, sem_r),
            pltpu.async_copy(rm_hbm.at[b], rm_v, sem_r),
            pltpu.async_copy(pre_hbm.at[b], pre_v, sem_r),
        ]
        for d in group_a:
            d.wait()

        def gat_h(ij):
            return plsc.load_gather(h_v, [ij >> 7, ij & 127])

        def gat_off(cc, ij):
            return plsc.load_gather(off_v, [cc, ij >> 7, ij & 127])

        def a_body(i, carry):
            acc, accn = carry
            nv = i * 16 + la
            hs, ys, xs, ms = [], [], [], []
            for j in range(4):
                jv = czero + j
                ij = plsc.load_gather(aidx_v, [jv, nv])
                hs.append(gat_h(ij))
                ys.append(gat_off(czero, ij) + OFFY[j])
                xs.append(gat_off(cone, ij) + OFFX[j])
                ms.append(plsc.load_gather(am_v, [jv, nv]))
            hm = (hs[0] + hs[1] + hs[2] + hs[3]) * 0.25
            ym = (ys[0] + ys[1] + ys[2] + ys[3]) * 0.25
            xm = (xs[0] + xs[1] + xs[2] + xs[3]) * 0.25
            wb = jnp.exp(hm)
            for j in range(4):
                wa = jnp.exp(hs[j])
                v = _iou16(wa, ys[j], xs[j], wb, ym, xm)
                acc = acc + ms[j] * (1.0 - v)
                accn = accn + ms[j]
            return acc, accn

        aa, aan = lax.fori_loop(0, N // 16, a_body, (zero, zero))

        for d in group_r:
            d.wait()

        def r_body(i, carry):
            acc, accn = carry
            mv = i * 16 + la
            boxes = []
            for p in range(2):
                pv = czero + p
                hsum, ysum, xsum = None, None, None
                for j in range(4):
                    ij = plsc.load_gather(ridx_v, [pv, czero + j, mv])
                    hj = gat_h(ij)
                    yj = gat_off(czero, ij)
                    xj = gat_off(cone, ij)
                    hsum = hj if hsum is None else hsum + hj
                    ysum = yj if ysum is None else ysum + yj
                    xsum = xj if xsum is None else xsum + xj
                hm = hsum * 0.25
                ym = ysum * 0.25 + 0.5
                xm = xsum * 0.25 + 0.5
                if p == 1:
                    ym = ym + plsc.load_gather(pre_v, [czero, mv])
                    xm = xm + plsc.load_gather(pre_v, [cone, mv])
                boxes.append((jnp.exp(hm), ym, xm))
            v = _iou16(*boxes[0], *boxes[1])
            mr = plsc.load_gather(rm_v, [mv])
            return acc + mr * v, accn + mr

        rr, rrn = lax.fori_loop(0, M // 16, r_body, (zero, zero))

        stage_v[pl.ds(0, 16)] = aa
        stage_v[pl.ds(16, 16)] = aan
        stage_v[pl.ds(32, 16)] = rr
        stage_v[pl.ds(48, 16)] = rrn
        stage_v[pl.ds(64, 16)] = zero
        stage_v[pl.ds(80, 16)] = zero
        stage_v[pl.ds(96, 16)] = zero
        stage_v[pl.ds(112, 16)] = zero
        pltpu.sync_copy(stage_v, shared_v.at[s])

    plsc.subcore_barrier()

    @pl.when(s == 0)
    def _final():
        pltpu.sync_copy(shared_v, buf_v)
        sa, san, sr, srn = zero, zero, zero, zero
        for w2 in range(NBC):
            sa = sa + buf_v[w2, pl.ds(0, 16)]
            san = san + buf_v[w2, pl.ds(16, 16)]
            sr = sr + buf_v[w2, pl.ds(32, 16)]
            srn = srn + buf_v[w2, pl.ds(48, 16)]
        stage_v[pl.ds(0, 16)] = sa
        stage_v[pl.ds(16, 16)] = san
        stage_v[pl.ds(32, 16)] = sr
        stage_v[pl.ds(48, 16)] = srn
        stage_v[pl.ds(64, 16)] = zero
        stage_v[pl.ds(80, 16)] = zero
        stage_v[pl.ds(96, 16)] = zero
        stage_v[pl.ds(112, 16)] = zero
        pltpu.sync_copy(stage_v, out_hbm.at[c])


def _combine_body(p_ref, o_ref):
    x = p_ref[...]
    s_a = jnp.sum(x[:, 0:16])
    s_an = jnp.sum(x[:, 16:32])
    s_r = jnp.sum(x[:, 32:48])
    s_rn = jnp.sum(x[:, 48:64])
    o_ref[0, 0] = s_a / (s_an + 1e-4) + s_r / (s_rn + 1e-4)


def kernel(output_h, output_off, target_h, target_off, attract, repel,
           mask_attract, mask_repel, pre_off):
    del target_h, target_off
    aidx = jnp.transpose(attract, (0, 2, 1)).astype(jnp.int32)
    ridx = jnp.transpose(repel, (0, 2, 3, 1)).astype(jnp.int32)
    am = jnp.transpose(mask_attract, (0, 2, 1)).astype(F)
    rm = mask_repel.astype(F).reshape(B, M)
    pre = jnp.transpose(pre_off, (0, 2, 1))

    mesh = plsc.VectorSubcoreMesh(core_axis_name="c", subcore_axis_name="s")
    sc_call = functools.partial(
        pl.kernel,
        out_type=jax.ShapeDtypeStruct((2, 128), F),
        mesh=mesh,
        compiler_params=pltpu.CompilerParams(needs_layout_passes=False),
        scratch_types=[
            pltpu.VMEM((H, W), F),
            pltpu.VMEM((2, H, W), F),
            pltpu.VMEM((4, N), jnp.int32),
            pltpu.VMEM((2, 4, M), jnp.int32),
            pltpu.VMEM((4, N), F),
            pltpu.VMEM((M,), F),
            pltpu.VMEM((2, M), F),
            pltpu.VMEM((128,), F),
            pltpu.VMEM((NBC, 128), F),
            pltpu.VMEM_SHARED((NBC, 128), F),
            pltpu.SemaphoreType.DMA,
            pltpu.SemaphoreType.DMA,
        ],
    )(_sc_body)
    partials = sc_call(output_h, output_off, aidx, ridx, am, rm, pre)

    loss = pl.pallas_call(
        _combine_body,
        out_shape=jax.ShapeDtypeStruct((1, 1), F),
        out_specs=pl.BlockSpec(memory_space=pltpu.SMEM),
    )(partials)
    return loss[0, 0]


# speedup vs baseline: 1.1208x; 1.1208x over previous
import functools

import jax
import jax.numpy as jnp
from jax import lax
from jax.experimental import pallas as pl
from jax.experimental.pallas import tpu as pltpu
from jax.experimental.pallas import tpu_sc as plsc

B, H, W, N, M = 16, 128, 128, 512, 512
HW = H * W
NS = 16

F = jnp.float32
C041 = 0.41
OFFY = (0.0, 1.0, 0.0, 1.0)
OFFX = (0.0, 0.0, 1.0, 1.0)


def _iou16(w0, y0, x0, w1, y1, x1):
    a0 = (w0 * w0) * C041
    a1 = (w1 * w1) * C041
    ay0 = y0 - w0 * 0.5
    ay1 = y0 + w0 * 0.5
    ax0 = x0 - (C041 * w0) * 0.5
    ax1 = x0 + (C041 * w0) * 0.5
    by0 = y1 - w1 * 0.5
    by1 = y1 + w1 * 0.5
    bx0 = x1 - (C041 * w1) * 0.5
    bx1 = x1 + (C041 * w1) * 0.5
    iy = jnp.maximum(jnp.minimum(ay1, by1) - jnp.maximum(ay0, by0), 0.0)
    ix = jnp.maximum(jnp.minimum(ax1, bx1) - jnp.maximum(ax0, bx0), 0.0)
    inter = iy * ix
    union = a0 + a1 - inter
    return inter / (union + 1e-6)


def _sc_body(h_hbm, off_hbm, aidx_hbm, ridx_hbm, am_hbm, rm_hbm, pre_hbm,
             out_hbm, h_v, off_v, aidx_v, ridx_v, am_v, rm_v, pre_v, stage_v,
             buf_v, shared_v, sem_a, sem_r):
    s = lax.axis_index("s")
    b = s

    group_a = [
        pltpu.async_copy(h_hbm.at[b, 0], h_v, sem_a),
        pltpu.async_copy(off_hbm.at[b], off_v, sem_a),
        pltpu.async_copy(aidx_hbm.at[b], aidx_v, sem_a),
        pltpu.async_copy(am_hbm.at[b], am_v, sem_a),
    ]
    group_r = [
        pltpu.async_copy(ridx_hbm.at[b], ridx_v, sem_r),
        pltpu.async_copy(rm_hbm.at[b], rm_v, sem_r),
        pltpu.async_copy(pre_hbm.at[b], pre_v, sem_r),
    ]
    for d in group_a:
        d.wait()

    la = lax.iota(jnp.int32, 16)
    zero = jnp.zeros((16,), F)
    czero = jnp.zeros((16,), jnp.int32)
    cone = czero + 1

    def gat_h(ij):
        return plsc.load_gather(h_v, [ij >> 7, ij & 127])

    def gat_off(cc, ij):
        return plsc.load_gather(off_v, [cc, ij >> 7, ij & 127])

    def a_body(i, carry):
        acc, accn = carry
        nv = i * 16 + la
        hs, ys, xs, ms = [], [], [], []
        for j in range(4):
            jv = czero + j
            ij = plsc.load_gather(aidx_v, [jv, nv])
            hs.append(gat_h(ij))
            ys.append(gat_off(czero, ij) + OFFY[j])
            xs.append(gat_off(cone, ij) + OFFX[j])
            ms.append(plsc.load_gather(am_v, [jv, nv]))
        hm = (hs[0] + hs[1] + hs[2] + hs[3]) * 0.25
        ym = (ys[0] + ys[1] + ys[2] + ys[3]) * 0.25
        xm = (xs[0] + xs[1] + xs[2] + xs[3]) * 0.25
        wb = jnp.exp(hm)
        for j in range(4):
            wa = jnp.exp(hs[j])
            v = _iou16(wa, ys[j], xs[j], wb, ym, xm)
            acc = acc + ms[j] * (1.0 - v)
            accn = accn + ms[j]
        return acc, accn

    acc_a, acc_an = lax.fori_loop(0, N // 16, a_body, (zero, zero))

    for d in group_r:
        d.wait()

    def r_body(i, carry):
        acc, accn = carry
        mv = i * 16 + la
        boxes = []
        for p in range(2):
            pv = czero + p
            hsum, ysum, xsum = None, None, None
            for j in range(4):
                ij = plsc.load_gather(ridx_v, [pv, czero + j, mv])
                hj = gat_h(ij)
                yj = gat_off(czero, ij)
                xj = gat_off(cone, ij)
                hsum = hj if hsum is None else hsum + hj
                ysum = yj if ysum is None else ysum + yj
                xsum = xj if xsum is None else xsum + xj
            hm = hsum * 0.25
            ym = ysum * 0.25 + 0.5
            xm = xsum * 0.25 + 0.5
            if p == 1:
                ym = ym + plsc.load_gather(pre_v, [czero, mv])
                xm = xm + plsc.load_gather(pre_v, [cone, mv])
            boxes.append((jnp.exp(hm), ym, xm))
        v = _iou16(*boxes[0], *boxes[1])
        mr = plsc.load_gather(rm_v, [mv])
        return acc + mr * v, accn + mr

    acc_r, acc_rn = lax.fori_loop(0, M // 16, r_body, (zero, zero))

    stage_v[pl.ds(0, 16)] = acc_a
    stage_v[pl.ds(16, 16)] = acc_an
    stage_v[pl.ds(32, 16)] = acc_r
    stage_v[pl.ds(48, 16)] = acc_rn
    stage_v[pl.ds(64, 16)] = zero
    stage_v[pl.ds(80, 16)] = zero
    stage_v[pl.ds(96, 16)] = zero
    stage_v[pl.ds(112, 16)] = zero
    pltpu.sync_copy(stage_v, shared_v.at[s])
    plsc.subcore_barrier()

    @pl.when(s == 0)
    def _final():
        pltpu.sync_copy(shared_v, buf_v)
        sa, san, sr, srn = zero, zero, zero, zero
        for w2 in range(NS):
            sa = sa + buf_v[w2, pl.ds(0, 16)]
            san = san + buf_v[w2, pl.ds(16, 16)]
            sr = sr + buf_v[w2, pl.ds(32, 16)]
            srn = srn + buf_v[w2, pl.ds(48, 16)]
        sav = zero + jnp.sum(sa)
        sanv = zero + jnp.sum(san)
        srv = zero + jnp.sum(sr)
        srnv = zero + jnp.sum(srn)
        stage_v[pl.ds(0, 16)] = sav / (sanv + 1e-4) + srv / (srnv + 1e-4)
        pltpu.sync_copy(stage_v.at[pl.ds(0, 8)], out_hbm)


def kernel(output_h, output_off, target_h, target_off, attract, repel,
           mask_attract, mask_repel, pre_off):
    del target_h, target_off
    aidx = jnp.transpose(attract, (0, 2, 1)).astype(jnp.int32)
    ridx = jnp.transpose(repel, (0, 2, 3, 1)).astype(jnp.int32)
    am = jnp.transpose(mask_attract, (0, 2, 1)).astype(F)
    rm = mask_repel.astype(F).reshape(B, M)
    pre = jnp.transpose(pre_off, (0, 2, 1))

    mesh = plsc.VectorSubcoreMesh(core_axis_name="c", subcore_axis_name="s",
                                  num_cores=1)
    sc_call = functools.partial(
        pl.kernel,
        out_type=jax.ShapeDtypeStruct((8,), F),
        mesh=mesh,
        compiler_params=pltpu.CompilerParams(needs_layout_passes=False),
        scratch_types=[
            pltpu.VMEM((H, W), F),
            pltpu.VMEM((2, H, W), F),
            pltpu.VMEM((4, N), jnp.int32),
            pltpu.VMEM((2, 4, M), jnp.int32),
            pltpu.VMEM((4, N), F),
            pltpu.VMEM((M,), F),
            pltpu.VMEM((2, M), F),
            pltpu.VMEM((128,), F),
            pltpu.VMEM((NS, 128), F),
            pltpu.VMEM_SHARED((NS, 128), F),
            pltpu.SemaphoreType.DMA,
            pltpu.SemaphoreType.DMA,
        ],
    )(_sc_body)
    out = sc_call(output_h, output_off, aidx, ridx, am, rm, pre)
    return out[0]
